# baseline (speedup 1.0000x reference)
import functools

import jax
import jax.numpy as jnp
from jax import lax
from jax.experimental import pallas as pl
from jax.experimental.pallas import tpu as pltpu
from jax.experimental.pallas import tpu_sc as plsc

N_NODES = 100000
D = 128
N_CLASS = 8
MIN_MATCH = 2
N_EXPANSION = 64

_BLK = 2048

_E = 3200000
_TOTAL = 8192
_NPAD = 100352
_NPACK = _NPAD // 4
_NPT = _NPAD // 16
_WPT = _NPT // 4
_SPT = _TOTAL // 16
_EPC = _E // 2
_EPT = _EPC // 16
_CHUNK = 2048
_NFULL = _EPT // _CHUNK
_TAILV = (_EPT - _NFULL * _CHUNK) // 16


def _sc_body(src_hbm, dst_hbm, seeds_hbm, segs_hbm, out_hbm,
             shc, shm, seedsb, segsb, vals2d, maskstage,
             srcb, dstb, widxb, wordb, cand_d, cand_m, idxb):
    c = lax.axis_index("c")
    s = lax.axis_index("s")
    iota = lax.iota(jnp.int32, 16)
    zero16f = jnp.zeros((16,), jnp.float32)
    zero16i = jnp.zeros((16,), jnp.int32)
    ones16f = jnp.ones((16,), jnp.float32)
    node_base = s * _NPT

    def _zrow(i, _):
        plsc.store_scatter(vals2d, [zero16i + i, iota], zero16f)
        return 0
    lax.fori_loop(0, 512, _zrow, 0)

    def _zc(k, _):
        base = pl.multiple_of(node_base + k * 512, 128)
        pltpu.sync_copy(vals2d, shc.at[pl.ds(base, 512)])
        return 0
    lax.fori_loop(0, _NPT // 512, _zc, 0)
    base128 = pl.multiple_of(node_base + (_NPT // 512) * 512, 128)
    pltpu.sync_copy(vals2d.at[pl.ds(0, _NPT % 512)],
                    shc.at[pl.ds(base128, _NPT % 512)])
    plsc.subcore_barrier()

    pltpu.sync_copy(seeds_hbm.at[pl.ds(s * _SPT, _SPT)], seedsb)
    pltpu.sync_copy(segs_hbm.at[pl.ds(s * _SPT, _SPT)], segsb)

    def _srow(i, _):
        seg = segsb[pl.ds(i * 16, 16)]
        plsc.store_scatter(vals2d, [i * 16 + iota, seg + 8], ones16f)
        return 0
    lax.fori_loop(0, _SPT // 16, _srow, 0)
    pltpu.sync_copy(vals2d, shc.at[seedsb], add=True)
    plsc.subcore_barrier()

    def _group(args):
        k, g = args
        for v in range(4):
            rows = g * 64 + v * 16 + iota
            mb = zero16i
            for cc in range(N_CLASS):
                x = plsc.load_gather(vals2d, [rows, zero16i + (8 + cc)])
                mb = mb | jnp.where(x > 0.0, jnp.int32(1 << cc), jnp.int32(0))
            idxb[pl.ds(v * 16, 16)] = mb
        w = zero16i
        for j in range(4):
            mj = plsc.load_gather(idxb, [iota * 4 + j])
            w = w | lax.shift_left(mj, 8 * j)
        maskstage[pl.ds((k * 8 + g) * 16, 16)] = w

    def _mchunk(k, _):
        base = pl.multiple_of(node_base + k * 512, 128)
        pltpu.sync_copy(shc.at[pl.ds(base, 512)], vals2d)

        def _g(g, _2):
            _group((k, g))
            return 0
        lax.fori_loop(0, 8, _g, 0)
        return 0
    lax.fori_loop(0, _NPT // 512, _mchunk, 0)
    pltpu.sync_copy(shc.at[pl.ds(base128, 128)], vals2d.at[pl.ds(0, 128)])

    def _gt(g, _):
        _group((_NPT // 512, g))
        return 0
    lax.fori_loop(0, 2, _gt, 0)
    pltpu.sync_copy(maskstage, shm.at[pl.ds(s * _WPT, _WPT)])
    plsc.subcore_barrier()

    lax.fori_loop(0, 512, _zrow, 0)

    def _build_and_fire():
        def _bv(i, _):
            mbv = cand_m[pl.ds(i * 16, 16)]
            rows = i * 16 + iota
            for cc in range(N_CLASS):
                bit = lax.shift_right_logical(mbv, cc) & 1
                plsc.store_scatter(vals2d, [rows, zero16i + cc],
                                   bit.astype(jnp.float32))
            idxb[pl.ds(i * 16, 16)] = cand_d[pl.ds(i * 16, 16)]
            return 0
        lax.fori_loop(0, 32, _bv, 0)
        pltpu.sync_copy(vals2d, shc.at[idxb], add=True)

    def _compact_step(i, cur):
        sv = srcb[pl.ds(i * 16, 16)]
        dv = dstb[pl.ds(i * 16, 16)]
        w = wordb[pl.ds(i * 16, 16)]
        sh = lax.shift_left(sv & 3, 3)
        mb = lax.shift_right_logical(w, sh) & 0xFF
        pred = mb != 0
        plsc.store_compressed(cand_d.at[pl.ds(cur, 16)], dv, mask=pred)
        plsc.store_compressed(cand_m.at[pl.ds(cur, 16)], mb, mask=pred)
        cur = cur + jnp.sum(pred.astype(jnp.int32))
        do_flush = cur >= 512

        @pl.when(do_flush)
        def _():
            _build_and_fire()
            ov = cand_d[pl.ds(512, 16)]
            om = cand_m[pl.ds(512, 16)]
            cand_d[pl.ds(0, 16)] = ov
            cand_m[pl.ds(0, 16)] = om
        return jnp.where(do_flush, cur - 512, cur)

    edge_base = c * _EPC + s * _EPT

    def _widx(i, _):
        widxb[pl.ds(i * 16, 16)] = lax.shift_right_logical(
            srcb[pl.ds(i * 16, 16)], 2)
        return 0

    def _chunk(k, cur):
        base = pl.multiple_of(edge_base + k * _CHUNK, 8)
        pltpu.sync_copy(src_hbm.at[pl.ds(base, _CHUNK)], srcb)
        pltpu.sync_copy(dst_hbm.at[pl.ds(base, _CHUNK)], dstb)
        lax.fori_loop(0, _CHUNK // 16, _widx, 0)
        pltpu.sync_copy(shm.at[widxb], wordb)
        return lax.fori_loop(0, _CHUNK // 16, _compact_step, cur)
    cur = lax.fori_loop(0, _NFULL, _chunk, 0)

    tbase = pl.multiple_of(edge_base + _NFULL * _CHUNK, 8)
    pltpu.sync_copy(src_hbm.at[pl.ds(tbase, _TAILV * 16)],
                    srcb.at[pl.ds(0, _TAILV * 16)])
    pltpu.sync_copy(dst_hbm.at[pl.ds(tbase, _TAILV * 16)],
                    dstb.at[pl.ds(0, _TAILV * 16)])
    lax.fori_loop(0, _TAILV, _widx, 0)
    pltpu.sync_copy(shm.at[widxb.at[pl.ds(0, _TAILV * 16)]],
                    wordb.at[pl.ds(0, _TAILV * 16)])
    cur = lax.fori_loop(0, _TAILV, _compact_step, cur)

    cand_d[pl.ds(cur, 16)] = N_NODES + ((iota + cur) & 255)
    cand_m[pl.ds(cur, 16)] = zero16i

    def _clr(i, _):
        pos = i * 16 + iota
        vd = cand_d[pl.ds(i * 16, 16)]
        vm = cand_m[pl.ds(i * 16, 16)]
        stale = pos >= cur + 16
        cand_d[pl.ds(i * 16, 16)] = jnp.where(stale, N_NODES + (pos & 255), vd)
        cand_m[pl.ds(i * 16, 16)] = jnp.where(stale, 0, vm)
        return 0
    lax.fori_loop(0, 32, _clr, 0)
    _build_and_fire()

    plsc.subcore_barrier()
    obase = pl.multiple_of(c * _NPAD + node_base, 128)
    nb = pl.multiple_of(node_base, 128)
    pltpu.sync_copy(shc.at[pl.ds(nb, _NPT)], out_hbm.at[pl.ds(obase, _NPT)])


def _sc_counts(src, dst, seeds, segs):
    mesh = plsc.VectorSubcoreMesh(core_axis_name="c", subcore_axis_name="s")
    f = pl.kernel(
        _sc_body,
        out_type=jax.ShapeDtypeStruct((2 * _NPAD, 16), jnp.float32),
        mesh=mesh,
        compiler_params=pltpu.CompilerParams(needs_layout_passes=False,
                                             use_tc_tiling_on_sc=False),
        scratch_types=[
            pltpu.VMEM_SHARED((_NPAD, 16), jnp.float32),
            pltpu.VMEM_SHARED((_NPACK,), jnp.int32),
            pltpu.VMEM((_SPT,), jnp.int32),
            pltpu.VMEM((_SPT,), jnp.int32),
            pltpu.VMEM((512, 16), jnp.float32),
            pltpu.VMEM((_WPT,), jnp.int32),
            pltpu.VMEM((_CHUNK,), jnp.int32),
            pltpu.VMEM((_CHUNK,), jnp.int32),
            pltpu.VMEM((_CHUNK,), jnp.int32),
            pltpu.VMEM((_CHUNK,), jnp.int32),
            pltpu.VMEM((528,), jnp.int32),
            pltpu.VMEM((528,), jnp.int32),
            pltpu.VMEM((512,), jnp.int32),
        ],
    )
    return f(src, dst, seeds, segs)


def _score_body(p_ref, es_ref, cnt_ref, out_ref):
    i = pl.program_id(0)
    p = p_ref[...]
    e = es_ref[...]
    s = lax.dot_general(p, e, (((1,), (1,)), ((), ())),
                        preferred_element_type=jnp.float32)
    cp = cnt_ref[...]
    m = cp[0] + cp[1]
    seed = jnp.sum(m[:, N_CLASS:], axis=1, keepdims=True)
    validf = jnp.where(
        (m[:, :N_CLASS] >= jnp.float32(MIN_MATCH)) & (seed <= 0.0), 1.0, 0.0)
    r = lax.broadcasted_iota(jnp.int32, (N_CLASS, N_CLASS), 0)
    cc = lax.broadcasted_iota(jnp.int32, (N_CLASS, N_CLASS), 1)
    eye = (r == cc).astype(jnp.float32)
    vt = lax.dot_general(eye, validf, (((1,), (1,)), ((), ())),
                         preferred_element_type=jnp.float32)
    col = i * _BLK + lax.broadcasted_iota(jnp.int32, (N_CLASS, _BLK), 1)
    keep = (vt > 0.5) & (col < N_NODES)
    sm = jnp.where(keep, s, jnp.float32(-1e9))
    out_ref[...] = sm


def _masked_scores(p, es, counts_part):
    grid = _NPAD // _BLK
    return pl.pallas_call(
        _score_body,
        grid=(grid,),
        in_specs=[
            pl.BlockSpec((N_CLASS, D), lambda i: (0, 0)),
            pl.BlockSpec((_BLK, D), lambda i: (i, 0)),
            pl.BlockSpec((2, _BLK, 16), lambda i: (0, i, 0)),
        ],
        out_specs=pl.BlockSpec((N_CLASS, _BLK), lambda i: (0, i)),
        out_shape=jax.ShapeDtypeStruct((N_CLASS, _NPAD), jnp.float32),
    )(p, es, counts_part)


_NBLKS = _NPAD // D


def _topk_body(s_ref, tv_ref, ti_ref, cm_ref):
    neginf = jnp.float32(-jnp.inf)

    for k in range(_NBLKS // D):
        chunk = s_ref[:, pl.ds(k * D * D, D * D)]
        cm_ref[:, pl.ds(k * D, D)] = jnp.max(
            chunk.reshape(N_CLASS, D, D), axis=2)
    _tail = _NBLKS % D
    chunk = s_ref[:, pl.ds((_NBLKS // D) * D * D, _tail * D)]
    cm_ref[:, pl.ds((_NBLKS // D) * D, _tail)] = jnp.max(
        chunk.reshape(N_CLASS, _tail, D), axis=2)

    def _iter(it, _):
        for c in range(N_CLASS):
            crow = cm_ref[c, :]
            mval = jnp.max(crow)
            biota = lax.iota(jnp.int32, _NBLKS)
            bidx = jnp.min(jnp.where(crow == mval, biota, _NBLKS))
            boff = pl.multiple_of(bidx * D, D)
            blk = s_ref[c, pl.ds(boff, D)]
            liota = lax.iota(jnp.int32, D)
            lidx = jnp.min(jnp.where(blk == mval, liota, D))
            kio = lax.iota(jnp.int32, N_EXPANSION)
            tv_ref[c, :] = jnp.where(kio == it, mval, tv_ref[c, :])
            ti_ref[c, :] = jnp.where(kio == it, bidx * D + lidx, ti_ref[c, :])
            nblk = jnp.where(liota == lidx, neginf, blk)
            s_ref[c, pl.ds(boff, D)] = nblk
            cm_ref[c, :] = jnp.where(biota == bidx, jnp.max(nblk), crow)
        return 0
    lax.fori_loop(0, N_EXPANSION, _iter, 0)


def _topk(scores):
    return pl.pallas_call(
        _topk_body,
        grid=(1,),
        in_specs=[
            pl.BlockSpec((N_CLASS, _NPAD), lambda i: (0, 0)),
        ],
        out_specs=[
            pl.BlockSpec((N_CLASS, N_EXPANSION), lambda i: (0, 0)),
            pl.BlockSpec((N_CLASS, N_EXPANSION), lambda i: (0, 0)),
        ],
        out_shape=[
            jax.ShapeDtypeStruct((N_CLASS, N_EXPANSION), jnp.float32),
            jax.ShapeDtypeStruct((N_CLASS, N_EXPANSION), jnp.int32),
        ],
        scratch_shapes=[pltpu.VMEM((N_CLASS, _NBLKS), jnp.float32)],
    )(scores)


@jax.jit
def kernel(es, W_in, b_in, Wx, Wh, b_gru, W_out, edge_index, flat_seeds,
           cu_seqlens):
    n, d = es.shape
    n_class = cu_seqlens.shape[0] - 1
    total = flat_seeds.shape[0]

    seg_ids = jnp.searchsorted(
        cu_seqlens, jnp.arange(total, dtype=cu_seqlens.dtype),
        side="right").astype(jnp.int32) - 1

    gathered = es[flat_seeds]
    sums = jax.ops.segment_sum(gathered, seg_ids, num_segments=n_class)
    lengths = (cu_seqlens[1:] - cu_seqlens[:-1]).astype(jnp.float32)
    denom = jnp.clip(lengths, 1.0, None)[:, None]
    pooled = sums / denom
    inp = jnp.tanh(pooled @ W_in + b_in)
    gx = inp @ Wx + b_gru
    z = jax.nn.sigmoid(gx[:, :d])
    hnew = (1.0 - z) * jnp.tanh(gx[:, 2 * d:])
    p = hnew @ W_out

    counts_part = _sc_counts(edge_index[0], edge_index[1], flat_seeds,
                             seg_ids).reshape(2, _NPAD, 16)

    scores = _masked_scores(p, es, counts_part)
    topv, topi = _topk(scores)
    return (topv, topi)


# speedup vs baseline: 1.1496x; 1.1496x over previous
import functools

import jax
import jax.numpy as jnp
from jax import lax
from jax.experimental import pallas as pl
from jax.experimental.pallas import tpu as pltpu
from jax.experimental.pallas import tpu_sc as plsc

N_NODES = 100000
D = 128
N_CLASS = 8
MIN_MATCH = 2
N_EXPANSION = 64

_BLK = 2048

_E = 3200000
_TOTAL = 8192
_NPAD = 100352
_NPACK = _NPAD // 4
_NPT = _NPAD // 16
_WPT = _NPT // 4
_SPT = _TOTAL // 16
_EPC = _E // 2
_EPT = _EPC // 16
_CHUNK = 2048
_NFULL = _EPT // _CHUNK
_TAILV = (_EPT - _NFULL * _CHUNK) // 16


def _sc_body(src_hbm, dst_hbm, seeds_hbm, segs_hbm, out_hbm,
             shc, shm, seedsb, segsb, vals2d, maskstage,
             srcb, dstb, widxb, wordb, cand_d, cand_m, idxb):
    c = lax.axis_index("c")
    s = lax.axis_index("s")
    iota = lax.iota(jnp.int32, 16)
    zero16f = jnp.zeros((16,), jnp.float32)
    zero16i = jnp.zeros((16,), jnp.int32)
    ones16f = jnp.ones((16,), jnp.float32)
    node_base = s * _NPT

    def _zrow(i, _):
        plsc.store_scatter(vals2d, [zero16i + i, iota], zero16f)
        return 0
    lax.fori_loop(0, 512, _zrow, 0)

    def _zc(k, _):
        base = pl.multiple_of(node_base + k * 512, 128)
        pltpu.sync_copy(vals2d, shc.at[pl.ds(base, 512)])
        return 0
    lax.fori_loop(0, _NPT // 512, _zc, 0)
    base128 = pl.multiple_of(node_base + (_NPT // 512) * 512, 128)
    pltpu.sync_copy(vals2d.at[pl.ds(0, _NPT % 512)],
                    shc.at[pl.ds(base128, _NPT % 512)])
    plsc.subcore_barrier()

    pltpu.sync_copy(seeds_hbm.at[pl.ds(s * _SPT, _SPT)], seedsb)
    pltpu.sync_copy(segs_hbm.at[pl.ds(s * _SPT, _SPT)], segsb)

    def _srow(i, _):
        seg = segsb[pl.ds(i * 16, 16)]
        plsc.store_scatter(vals2d, [i * 16 + iota, seg + 8], ones16f)
        return 0
    lax.fori_loop(0, _SPT // 16, _srow, 0)
    pltpu.sync_copy(vals2d, shc.at[seedsb], add=True)
    plsc.subcore_barrier()

    def _group(args):
        k, g = args
        for v in range(4):
            rows = g * 64 + v * 16 + iota
            mb = zero16i
            for cc in range(N_CLASS):
                x = plsc.load_gather(vals2d, [rows, zero16i + (8 + cc)])
                mb = mb | jnp.where(x > 0.0, jnp.int32(1 << cc), jnp.int32(0))
            idxb[pl.ds(v * 16, 16)] = mb
        w = zero16i
        for j in range(4):
            mj = plsc.load_gather(idxb, [iota * 4 + j])
            w = w | lax.shift_left(mj, 8 * j)
        maskstage[pl.ds((k * 8 + g) * 16, 16)] = w

    def _mchunk(k, _):
        base = pl.multiple_of(node_base + k * 512, 128)
        pltpu.sync_copy(shc.at[pl.ds(base, 512)], vals2d)

        def _g(g, _2):
            _group((k, g))
            return 0
        lax.fori_loop(0, 8, _g, 0)
        return 0
    lax.fori_loop(0, _NPT // 512, _mchunk, 0)
    pltpu.sync_copy(shc.at[pl.ds(base128, 128)], vals2d.at[pl.ds(0, 128)])

    def _gt(g, _):
        _group((_NPT // 512, g))
        return 0
    lax.fori_loop(0, 2, _gt, 0)
    pltpu.sync_copy(maskstage, shm.at[pl.ds(s * _WPT, _WPT)])
    plsc.subcore_barrier()

    lax.fori_loop(0, 512, _zrow, 0)

    def _build_and_fire():
        def _bv(i, _):
            mbv = cand_m[pl.ds(i * 16, 16)]
            rows = i * 16 + iota
            for cc in range(N_CLASS):
                bit = lax.shift_right_logical(mbv, cc) & 1
                plsc.store_scatter(vals2d, [rows, zero16i + cc],
                                   bit.astype(jnp.float32))
            idxb[pl.ds(i * 16, 16)] = cand_d[pl.ds(i * 16, 16)]
            return 0
        lax.fori_loop(0, 32, _bv, 0)
        pltpu.sync_copy(vals2d, shc.at[idxb], add=True)

    def _compact_step(i, cur):
        sv = srcb[pl.ds(i * 16, 16)]
        dv = dstb[pl.ds(i * 16, 16)]
        w = wordb[pl.ds(i * 16, 16)]
        sh = lax.shift_left(sv & 3, 3)
        mb = lax.shift_right_logical(w, sh) & 0xFF
        pred = mb != 0
        plsc.store_compressed(cand_d.at[pl.ds(cur, 16)], dv, mask=pred)
        plsc.store_compressed(cand_m.at[pl.ds(cur, 16)], mb, mask=pred)
        cur = cur + jnp.sum(pred.astype(jnp.int32))
        do_flush = cur >= 512

        @pl.when(do_flush)
        def _():
            _build_and_fire()
            ov = cand_d[pl.ds(512, 16)]
            om = cand_m[pl.ds(512, 16)]
            cand_d[pl.ds(0, 16)] = ov
            cand_m[pl.ds(0, 16)] = om
        return jnp.where(do_flush, cur - 512, cur)

    edge_base = c * _EPC + s * _EPT

    def _widx(i, _):
        widxb[pl.ds(i * 16, 16)] = lax.shift_right_logical(
            srcb[pl.ds(i * 16, 16)], 2)
        return 0

    def _chunk(k, cur):
        base = pl.multiple_of(edge_base + k * _CHUNK, 8)
        pltpu.sync_copy(src_hbm.at[pl.ds(base, _CHUNK)], srcb)
        pltpu.sync_copy(dst_hbm.at[pl.ds(base, _CHUNK)], dstb)
        lax.fori_loop(0, _CHUNK // 16, _widx, 0)
        pltpu.sync_copy(shm.at[widxb], wordb)
        return lax.fori_loop(0, _CHUNK // 16, _compact_step, cur)
    cur = lax.fori_loop(0, _NFULL, _chunk, 0)

    tbase = pl.multiple_of(edge_base + _NFULL * _CHUNK, 8)
    pltpu.sync_copy(src_hbm.at[pl.ds(tbase, _TAILV * 16)],
                    srcb.at[pl.ds(0, _TAILV * 16)])
    pltpu.sync_copy(dst_hbm.at[pl.ds(tbase, _TAILV * 16)],
                    dstb.at[pl.ds(0, _TAILV * 16)])
    lax.fori_loop(0, _TAILV, _widx, 0)
    pltpu.sync_copy(shm.at[widxb.at[pl.ds(0, _TAILV * 16)]],
                    wordb.at[pl.ds(0, _TAILV * 16)])
    cur = lax.fori_loop(0, _TAILV, _compact_step, cur)

    cand_d[pl.ds(cur, 16)] = N_NODES + ((iota + cur) & 255)
    cand_m[pl.ds(cur, 16)] = zero16i

    def _clr(i, _):
        pos = i * 16 + iota
        vd = cand_d[pl.ds(i * 16, 16)]
        vm = cand_m[pl.ds(i * 16, 16)]
        stale = pos >= cur + 16
        cand_d[pl.ds(i * 16, 16)] = jnp.where(stale, N_NODES + (pos & 255), vd)
        cand_m[pl.ds(i * 16, 16)] = jnp.where(stale, 0, vm)
        return 0
    lax.fori_loop(0, 32, _clr, 0)
    _build_and_fire()

    plsc.subcore_barrier()
    obase = pl.multiple_of(c * _NPAD + node_base, 128)
    nb = pl.multiple_of(node_base, 128)
    pltpu.sync_copy(shc.at[pl.ds(nb, _NPT)], out_hbm.at[pl.ds(obase, _NPT)])


def _sc_counts(src, dst, seeds, segs):
    mesh = plsc.VectorSubcoreMesh(core_axis_name="c", subcore_axis_name="s")
    f = pl.kernel(
        _sc_body,
        out_type=jax.ShapeDtypeStruct((2 * _NPAD, 16), jnp.float32),
        mesh=mesh,
        compiler_params=pltpu.CompilerParams(needs_layout_passes=False,
                                             use_tc_tiling_on_sc=False),
        scratch_types=[
            pltpu.VMEM_SHARED((_NPAD, 16), jnp.float32),
            pltpu.VMEM_SHARED((_NPACK,), jnp.int32),
            pltpu.VMEM((_SPT,), jnp.int32),
            pltpu.VMEM((_SPT,), jnp.int32),
            pltpu.VMEM((512, 16), jnp.float32),
            pltpu.VMEM((_WPT,), jnp.int32),
            pltpu.VMEM((_CHUNK,), jnp.int32),
            pltpu.VMEM((_CHUNK,), jnp.int32),
            pltpu.VMEM((_CHUNK,), jnp.int32),
            pltpu.VMEM((_CHUNK,), jnp.int32),
            pltpu.VMEM((528,), jnp.int32),
            pltpu.VMEM((528,), jnp.int32),
            pltpu.VMEM((512,), jnp.int32),
        ],
    )
    return f(src, dst, seeds, segs)


def _score_body(p_ref, es_ref, cnt_ref, cnt2_ref, out_ref):
    i = pl.program_id(0)
    p = p_ref[...]
    e = es_ref[...]
    s = lax.dot_general(p, e, (((1,), (1,)), ((), ())),
                        preferred_element_type=jnp.float32)
    m = cnt_ref[...] + cnt2_ref[...]
    seed = jnp.sum(m[:, N_CLASS:], axis=1, keepdims=True)
    validf = jnp.where(
        (m[:, :N_CLASS] >= jnp.float32(MIN_MATCH)) & (seed <= 0.0), 1.0, 0.0)
    r = lax.broadcasted_iota(jnp.int32, (N_CLASS, N_CLASS), 0)
    cc = lax.broadcasted_iota(jnp.int32, (N_CLASS, N_CLASS), 1)
    eye = (r == cc).astype(jnp.float32)
    vt = lax.dot_general(eye, validf, (((1,), (1,)), ((), ())),
                         preferred_element_type=jnp.float32)
    col = i * _BLK + lax.broadcasted_iota(jnp.int32, (N_CLASS, _BLK), 1)
    keep = (vt > 0.5) & (col < N_NODES)
    sm = jnp.where(keep, s, jnp.float32(-1e9))
    out_ref[...] = sm


def _masked_scores(p, es, counts_part):
    grid = _NPAD // _BLK
    return pl.pallas_call(
        _score_body,
        grid=(grid,),
        in_specs=[
            pl.BlockSpec((N_CLASS, D), lambda i: (0, 0)),
            pl.BlockSpec((_BLK, D), lambda i: (i, 0)),
            pl.BlockSpec((_BLK, 16), lambda i: (i, 0)),
            pl.BlockSpec((_BLK, 16), lambda i: (i + _NPAD // _BLK, 0)),
        ],
        out_specs=pl.BlockSpec((N_CLASS, _BLK), lambda i: (0, i)),
        out_shape=jax.ShapeDtypeStruct((N_CLASS, _NPAD), jnp.float32),
    )(p, es, counts_part, counts_part)


_NBLKS = _NPAD // D


def _topk_body(s_ref, tv_ref, ti_ref, cm_ref):
    neginf = jnp.float32(-jnp.inf)

    for k in range(_NBLKS // D):
        chunk = s_ref[:, pl.ds(k * D * D, D * D)]
        cm_ref[:, pl.ds(k * D, D)] = jnp.max(
            chunk.reshape(N_CLASS, D, D), axis=2)
    _tail = _NBLKS % D
    chunk = s_ref[:, pl.ds((_NBLKS // D) * D * D, _tail * D)]
    cm_ref[:, pl.ds((_NBLKS // D) * D, _tail)] = jnp.max(
        chunk.reshape(N_CLASS, _tail, D), axis=2)

    kio = lax.broadcasted_iota(jnp.int32, (N_CLASS, N_EXPANSION), 1)
    biota = lax.broadcasted_iota(jnp.int32, (N_CLASS, _NBLKS), 1)
    liota = lax.iota(jnp.int32, D)
    rows = lax.broadcasted_iota(jnp.int32, (N_CLASS, 1), 0)

    def _iter(it, _):
        cm = cm_ref[...]
        m8 = jnp.max(cm, axis=1, keepdims=True)
        bidx8 = jnp.min(jnp.where(cm == m8, biota, _NBLKS),
                        axis=1, keepdims=True)
        lidx_acc = jnp.zeros((N_CLASS, 1), jnp.int32)
        nm_acc = jnp.zeros((N_CLASS, 1), jnp.float32)
        for c in range(N_CLASS):
            bidx = jnp.max(jnp.where(rows == c, bidx8, 0))
            mc = jnp.max(jnp.where(rows == c, m8, neginf))
            boff = pl.multiple_of(bidx * D, D)
            blk = s_ref[c, pl.ds(boff, D)]
            lidx = jnp.min(jnp.where(blk == mc, liota, D))
            nblk = jnp.where(liota == lidx, neginf, blk)
            s_ref[c, pl.ds(boff, D)] = nblk
            lidx_acc = jnp.where(rows == c, lidx, lidx_acc)
            nm_acc = jnp.where(rows == c, jnp.max(nblk), nm_acc)
        tv_ref[...] = jnp.where(kio == it, m8, tv_ref[...])
        ti_ref[...] = jnp.where(kio == it, bidx8 * D + lidx_acc, ti_ref[...])
        cm_ref[...] = jnp.where(biota == bidx8, nm_acc, cm)
        return 0
    lax.fori_loop(0, N_EXPANSION, _iter, 0)


def _topk(scores):
    return pl.pallas_call(
        _topk_body,
        grid=(1,),
        in_specs=[
            pl.BlockSpec((N_CLASS, _NPAD), lambda i: (0, 0)),
        ],
        out_specs=[
            pl.BlockSpec((N_CLASS, N_EXPANSION), lambda i: (0, 0)),
            pl.BlockSpec((N_CLASS, N_EXPANSION), lambda i: (0, 0)),
        ],
        out_shape=[
            jax.ShapeDtypeStruct((N_CLASS, N_EXPANSION), jnp.float32),
            jax.ShapeDtypeStruct((N_CLASS, N_EXPANSION), jnp.int32),
        ],
        scratch_shapes=[pltpu.VMEM((N_CLASS, _NBLKS), jnp.float32)],
    )(scores)


@jax.jit
def kernel(es, W_in, b_in, Wx, Wh, b_gru, W_out, edge_index, flat_seeds,
           cu_seqlens):
    n, d = es.shape
    n_class = cu_seqlens.shape[0] - 1
    total = flat_seeds.shape[0]

    pos = jnp.arange(total, dtype=jnp.int32)
    seg_ids = jnp.sum(
        (pos[:, None] >= cu_seqlens[None, 1:n_class]).astype(jnp.int32),
        axis=1)

    gathered = es[flat_seeds]
    sums = jax.ops.segment_sum(gathered, seg_ids, num_segments=n_class)
    lengths = (cu_seqlens[1:] - cu_seqlens[:-1]).astype(jnp.float32)
    denom = jnp.clip(lengths, 1.0, None)[:, None]
    pooled = sums / denom
    inp = jnp.tanh(pooled @ W_in + b_in)
    gx = inp @ Wx + b_gru
    z = jax.nn.sigmoid(gx[:, :d])
    hnew = (1.0 - z) * jnp.tanh(gx[:, 2 * d:])
    p = hnew @ W_out

    counts_part = _sc_counts(edge_index[0], edge_index[1], flat_seeds,
                             seg_ids)

    scores = _masked_scores(p, es, counts_part)
    topv, topi = _topk(scores)
    return (topv, topi)


# speedup vs baseline: 1.4164x; 1.2320x over previous
import functools

import jax
import jax.numpy as jnp
from jax import lax
from jax.experimental import pallas as pl
from jax.experimental.pallas import tpu as pltpu
from jax.experimental.pallas import tpu_sc as plsc

N_NODES = 100000
D = 128
N_CLASS = 8
MIN_MATCH = 2
N_EXPANSION = 64

_BLK = 2048

_E = 3200000
_TOTAL = 8192
_NPAD = 100352
_NPACK = _NPAD // 4
_NPT = _NPAD // 16
_WPT = _NPT // 4
_SPT = _TOTAL // 16
_EPC = _E // 2
_EPT = _EPC // 16
_CHUNK = 2048
_NFULL = _EPT // _CHUNK
_TAILV = (_EPT - _NFULL * _CHUNK) // 16


def _sc_body(src_hbm, dst_hbm, seeds_hbm, segs_hbm, out_hbm,
             shc, shm, seedsb, segsb, vals2d, maskstage,
             srcb, dstb, widxb, wordb, cand_d, cand_m, idxb):
    c = lax.axis_index("c")
    s = lax.axis_index("s")
    iota = lax.iota(jnp.int32, 16)
    zero16f = jnp.zeros((16,), jnp.float32)
    zero16i = jnp.zeros((16,), jnp.int32)
    ones16f = jnp.ones((16,), jnp.float32)
    node_base = s * _NPT

    def _zrow(i, _):
        plsc.store_scatter(vals2d, [zero16i + i, iota], zero16f)
        return 0
    lax.fori_loop(0, 512, _zrow, 0)

    def _zc(k, _):
        base = pl.multiple_of(node_base + k * 512, 128)
        pltpu.sync_copy(vals2d, shc.at[pl.ds(base, 512)])
        return 0
    lax.fori_loop(0, _NPT // 512, _zc, 0)
    base128 = pl.multiple_of(node_base + (_NPT // 512) * 512, 128)
    pltpu.sync_copy(vals2d.at[pl.ds(0, _NPT % 512)],
                    shc.at[pl.ds(base128, _NPT % 512)])
    plsc.subcore_barrier()

    pltpu.sync_copy(seeds_hbm.at[pl.ds(s * _SPT, _SPT)], seedsb)
    pltpu.sync_copy(segs_hbm.at[pl.ds(s * _SPT, _SPT)], segsb)

    def _srow(i, _):
        seg = segsb[pl.ds(i * 16, 16)]
        plsc.store_scatter(vals2d, [i * 16 + iota, seg + 8], ones16f)
        return 0
    lax.fori_loop(0, _SPT // 16, _srow, 0)
    pltpu.sync_copy(vals2d, shc.at[seedsb], add=True)
    plsc.subcore_barrier()

    def _group(args):
        k, g = args
        for v in range(4):
            rows = g * 64 + v * 16 + iota
            mb = zero16i
            for cc in range(N_CLASS):
                x = plsc.load_gather(vals2d, [rows, zero16i + (8 + cc)])
                mb = mb | jnp.where(x > 0.0, jnp.int32(1 << cc), jnp.int32(0))
            idxb[pl.ds(v * 16, 16)] = mb
        w = zero16i
        for j in range(4):
            mj = plsc.load_gather(idxb, [iota * 4 + j])
            w = w | lax.shift_left(mj, 8 * j)
        maskstage[pl.ds((k * 8 + g) * 16, 16)] = w

    def _mchunk(k, _):
        base = pl.multiple_of(node_base + k * 512, 128)
        pltpu.sync_copy(shc.at[pl.ds(base, 512)], vals2d)

        def _g(g, _2):
            _group((k, g))
            return 0
        lax.fori_loop(0, 8, _g, 0)
        return 0
    lax.fori_loop(0, _NPT // 512, _mchunk, 0)
    pltpu.sync_copy(shc.at[pl.ds(base128, 128)], vals2d.at[pl.ds(0, 128)])

    def _gt(g, _):
        _group((_NPT // 512, g))
        return 0
    lax.fori_loop(0, 2, _gt, 0)
    pltpu.sync_copy(maskstage, shm.at[pl.ds(s * _WPT, _WPT)])
    plsc.subcore_barrier()

    lax.fori_loop(0, 512, _zrow, 0)

    def _build_and_fire():
        def _bv(i, _):
            mbv = cand_m[pl.ds(i * 16, 16)]
            rows = i * 16 + iota
            for cc in range(N_CLASS):
                bit = lax.shift_right_logical(mbv, cc) & 1
                plsc.store_scatter(vals2d, [rows, zero16i + cc],
                                   bit.astype(jnp.float32))
            idxb[pl.ds(i * 16, 16)] = cand_d[pl.ds(i * 16, 16)]
            return 0
        lax.fori_loop(0, 32, _bv, 0)
        pltpu.sync_copy(vals2d, shc.at[idxb], add=True)

    def _compact_step(i, cur):
        sv = srcb[pl.ds(i * 16, 16)]
        dv = dstb[pl.ds(i * 16, 16)]
        w = wordb[pl.ds(i * 16, 16)]
        sh = lax.shift_left(sv & 3, 3)
        mb = lax.shift_right_logical(w, sh) & 0xFF
        pred = mb != 0
        plsc.store_compressed(cand_d.at[pl.ds(cur, 16)], dv, mask=pred)
        plsc.store_compressed(cand_m.at[pl.ds(cur, 16)], mb, mask=pred)
        cur = cur + jnp.sum(pred.astype(jnp.int32))
        do_flush = cur >= 512

        @pl.when(do_flush)
        def _():
            _build_and_fire()
            ov = cand_d[pl.ds(512, 16)]
            om = cand_m[pl.ds(512, 16)]
            cand_d[pl.ds(0, 16)] = ov
            cand_m[pl.ds(0, 16)] = om
        return jnp.where(do_flush, cur - 512, cur)

    edge_base = c * _EPC + s * _EPT

    def _widx(i, _):
        widxb[pl.ds(i * 16, 16)] = lax.shift_right_logical(
            srcb[pl.ds(i * 16, 16)], 2)
        return 0

    def _chunk(k, cur):
        base = pl.multiple_of(edge_base + k * _CHUNK, 8)
        pltpu.sync_copy(src_hbm.at[pl.ds(base, _CHUNK)], srcb)
        pltpu.sync_copy(dst_hbm.at[pl.ds(base, _CHUNK)], dstb)
        lax.fori_loop(0, _CHUNK // 16, _widx, 0)
        pltpu.sync_copy(shm.at[widxb], wordb)
        return lax.fori_loop(0, _CHUNK // 16, _compact_step, cur)
    cur = lax.fori_loop(0, _NFULL, _chunk, 0)

    tbase = pl.multiple_of(edge_base + _NFULL * _CHUNK, 8)
    pltpu.sync_copy(src_hbm.at[pl.ds(tbase, _TAILV * 16)],
                    srcb.at[pl.ds(0, _TAILV * 16)])
    pltpu.sync_copy(dst_hbm.at[pl.ds(tbase, _TAILV * 16)],
                    dstb.at[pl.ds(0, _TAILV * 16)])
    lax.fori_loop(0, _TAILV, _widx, 0)
    pltpu.sync_copy(shm.at[widxb.at[pl.ds(0, _TAILV * 16)]],
                    wordb.at[pl.ds(0, _TAILV * 16)])
    cur = lax.fori_loop(0, _TAILV, _compact_step, cur)

    cand_d[pl.ds(cur, 16)] = N_NODES + ((iota + cur) & 255)
    cand_m[pl.ds(cur, 16)] = zero16i

    def _clr(i, _):
        pos = i * 16 + iota
        vd = cand_d[pl.ds(i * 16, 16)]
        vm = cand_m[pl.ds(i * 16, 16)]
        stale = pos >= cur + 16
        cand_d[pl.ds(i * 16, 16)] = jnp.where(stale, N_NODES + (pos & 255), vd)
        cand_m[pl.ds(i * 16, 16)] = jnp.where(stale, 0, vm)
        return 0
    lax.fori_loop(0, 32, _clr, 0)
    _build_and_fire()

    plsc.subcore_barrier()
    obase = pl.multiple_of(c * _NPAD + node_base, 128)
    nb = pl.multiple_of(node_base, 128)
    pltpu.sync_copy(shc.at[pl.ds(nb, _NPT)], out_hbm.at[pl.ds(obase, _NPT)])


def _sc_counts(src, dst, seeds, segs):
    mesh = plsc.VectorSubcoreMesh(core_axis_name="c", subcore_axis_name="s")
    f = pl.kernel(
        _sc_body,
        out_type=jax.ShapeDtypeStruct((2 * _NPAD, 16), jnp.float32),
        mesh=mesh,
        compiler_params=pltpu.CompilerParams(needs_layout_passes=False,
                                             use_tc_tiling_on_sc=False),
        scratch_types=[
            pltpu.VMEM_SHARED((_NPAD, 16), jnp.float32),
            pltpu.VMEM_SHARED((_NPACK,), jnp.int32),
            pltpu.VMEM((_SPT,), jnp.int32),
            pltpu.VMEM((_SPT,), jnp.int32),
            pltpu.VMEM((512, 16), jnp.float32),
            pltpu.VMEM((_WPT,), jnp.int32),
            pltpu.VMEM((_CHUNK,), jnp.int32),
            pltpu.VMEM((_CHUNK,), jnp.int32),
            pltpu.VMEM((_CHUNK,), jnp.int32),
            pltpu.VMEM((_CHUNK,), jnp.int32),
            pltpu.VMEM((528,), jnp.int32),
            pltpu.VMEM((528,), jnp.int32),
            pltpu.VMEM((512,), jnp.int32),
        ],
    )
    return f(src, dst, seeds, segs)


def _score_body(p_ref, es_ref, cnt_ref, cnt2_ref, out_ref):
    i = pl.program_id(0)
    p = p_ref[...]
    e = es_ref[...]
    s = lax.dot_general(p, e, (((1,), (1,)), ((), ())),
                        preferred_element_type=jnp.float32)
    m = cnt_ref[...] + cnt2_ref[...]
    seed = jnp.sum(m[:, N_CLASS:], axis=1, keepdims=True)
    validf = jnp.where(
        (m[:, :N_CLASS] >= jnp.float32(MIN_MATCH)) & (seed <= 0.0), 1.0, 0.0)
    r = lax.broadcasted_iota(jnp.int32, (N_CLASS, N_CLASS), 0)
    cc = lax.broadcasted_iota(jnp.int32, (N_CLASS, N_CLASS), 1)
    eye = (r == cc).astype(jnp.float32)
    vt = lax.dot_general(eye, validf, (((1,), (1,)), ((), ())),
                         preferred_element_type=jnp.float32)
    col = i * _BLK + lax.broadcasted_iota(jnp.int32, (N_CLASS, _BLK), 1)
    keep = (vt > 0.5) & (col < N_NODES)
    sm = jnp.where(keep, s, jnp.float32(-1e9))
    out_ref[...] = sm


def _masked_scores(p, es, counts_part):
    grid = _NPAD // _BLK
    return pl.pallas_call(
        _score_body,
        grid=(grid,),
        in_specs=[
            pl.BlockSpec((N_CLASS, D), lambda i: (0, 0)),
            pl.BlockSpec((_BLK, D), lambda i: (i, 0)),
            pl.BlockSpec((_BLK, 16), lambda i: (i, 0)),
            pl.BlockSpec((_BLK, 16), lambda i: (i + _NPAD // _BLK, 0)),
        ],
        out_specs=pl.BlockSpec((N_CLASS, _BLK), lambda i: (0, i)),
        out_shape=jax.ShapeDtypeStruct((N_CLASS, _NPAD), jnp.float32),
    )(p, es, counts_part, counts_part)


_NBLKS = _NPAD // D


def _topk_body(s_ref, tv_ref, ti_ref, cm_ref):
    neginf = jnp.float32(-jnp.inf)

    for k in range(_NBLKS // D):
        chunk = s_ref[:, pl.ds(k * D * D, D * D)]
        cm_ref[:, pl.ds(k * D, D)] = jnp.max(
            chunk.reshape(N_CLASS, D, D), axis=2)
    _tail = _NBLKS % D
    chunk = s_ref[:, pl.ds((_NBLKS // D) * D * D, _tail * D)]
    cm_ref[:, pl.ds((_NBLKS // D) * D, _tail)] = jnp.max(
        chunk.reshape(N_CLASS, _tail, D), axis=2)

    kio = lax.broadcasted_iota(jnp.int32, (N_CLASS, N_EXPANSION), 1)
    biota = lax.broadcasted_iota(jnp.int32, (N_CLASS, _NBLKS), 1)
    liota = lax.iota(jnp.int32, D)
    rows = lax.broadcasted_iota(jnp.int32, (N_CLASS, 1), 0)

    rows2d = lax.broadcasted_iota(jnp.int32, (N_CLASS, D), 0)
    liota2d = lax.broadcasted_iota(jnp.int32, (N_CLASS, D), 1)

    def _iter(it, _):
        cm = cm_ref[...]
        m8 = jnp.max(cm, axis=1, keepdims=True)
        bidx8 = jnp.min(jnp.where(cm == m8, biota, _NBLKS),
                        axis=1, keepdims=True)
        boffs = []
        t = jnp.zeros((N_CLASS, D), jnp.float32)
        for c in range(N_CLASS):
            bidx = jnp.max(jnp.where(rows == c, bidx8, 0))
            boff = pl.multiple_of(bidx * D, D)
            boffs.append(boff)
            blk = s_ref[c, pl.ds(boff, D)]
            t = jnp.where(rows2d == c, blk[None, :], t)
        lidx8 = jnp.min(jnp.where(t == m8, liota2d, D),
                        axis=1, keepdims=True)
        nt = jnp.where(liota2d == lidx8, neginf, t)
        nm8 = jnp.max(nt, axis=1, keepdims=True)
        for c in range(N_CLASS):
            s_ref[c, pl.ds(boffs[c], D)] = nt[c, :]
        tv_ref[...] = jnp.where(kio == it, m8, tv_ref[...])
        ti_ref[...] = jnp.where(kio == it, bidx8 * D + lidx8, ti_ref[...])
        cm_ref[...] = jnp.where(biota == bidx8, nm8, cm)
        return 0
    lax.fori_loop(0, N_EXPANSION, _iter, 0)


def _topk(scores):
    return pl.pallas_call(
        _topk_body,
        grid=(1,),
        in_specs=[
            pl.BlockSpec((N_CLASS, _NPAD), lambda i: (0, 0)),
        ],
        out_specs=[
            pl.BlockSpec((N_CLASS, N_EXPANSION), lambda i: (0, 0)),
            pl.BlockSpec((N_CLASS, N_EXPANSION), lambda i: (0, 0)),
        ],
        out_shape=[
            jax.ShapeDtypeStruct((N_CLASS, N_EXPANSION), jnp.float32),
            jax.ShapeDtypeStruct((N_CLASS, N_EXPANSION), jnp.int32),
        ],
        scratch_shapes=[pltpu.VMEM((N_CLASS, _NBLKS), jnp.float32)],
    )(scores)


@jax.jit
def kernel(es, W_in, b_in, Wx, Wh, b_gru, W_out, edge_index, flat_seeds,
           cu_seqlens):
    n, d = es.shape
    n_class = cu_seqlens.shape[0] - 1
    total = flat_seeds.shape[0]

    pos = jnp.arange(total, dtype=jnp.int32)
    seg_ids = jnp.sum(
        (pos[:, None] >= cu_seqlens[None, 1:n_class]).astype(jnp.int32),
        axis=1)

    gathered = es[flat_seeds]
    sums = jax.ops.segment_sum(gathered, seg_ids, num_segments=n_class)
    lengths = (cu_seqlens[1:] - cu_seqlens[:-1]).astype(jnp.float32)
    denom = jnp.clip(lengths, 1.0, None)[:, None]
    pooled = sums / denom
    inp = jnp.tanh(pooled @ W_in + b_in)
    gx = inp @ Wx + b_gru
    z = jax.nn.sigmoid(gx[:, :d])
    hnew = (1.0 - z) * jnp.tanh(gx[:, 2 * d:])
    p = hnew @ W_out

    counts_part = _sc_counts(edge_index[0], edge_index[1], flat_seeds,
                             seg_ids)

    scores = _masked_scores(p, es, counts_part)
    topv, topi = _topk(scores)
    return (topv, topi)
